# Initial kernel scaffold; baseline (speedup 1.0000x reference)
import jax
import jax.numpy as jnp
from jax.experimental import pallas as pl


def kernel(features, W1, b1, W_box, b_box, W_rpn, b_rpn):
    raise NotImplementedError("write your pallas kernel here")


# speedup vs baseline: 10.0690x; 10.0690x over previous
import functools

import jax
import jax.numpy as jnp
from jax import lax
from jax.experimental import pallas as pl
from jax.experimental.pallas import tpu as pltpu

_INTERPRET = False

_K = 12
_FH = _FW = 32
_P = _FH * _FW
_NB = _K * _P
_ROWS = _NB // 128
_IMGW = 512.0
_IMGH = 512.0
_NMS_T = 0.7
_MAXP = 300
_OH = _OW = 7
_NPIX = _MAXP * _OH * _OW
_NPAD = 14720
_BLK = 1472
_X0 = 7.5
_Y0 = 7.5
_SX = 16.0
_SY = 16.0

_HIGH = lax.Precision.HIGHEST


def _trunk_nms_body(xT_ref, w1_ref, b1_ref,
                    wb0_ref, wb1_ref, wb2_ref, wb3_ref,
                    bb0_ref, bb1_ref, bb2_ref, bb3_ref,
                    wr_ref, br_ref,
                    xa_ref, ya_ref, wa_ref, ha_ref,
                    out_ref, dbg_ref):
    baseT = jnp.dot(w1_ref[...], xT_ref[...],
                    preferred_element_type=jnp.float32)
    baseT = jnp.maximum(baseT + b1_ref[...], 0.0)

    def head(w_ref, b_ref):
        h = jnp.dot(w_ref[...], baseT, preferred_element_type=jnp.float32)
        return (h + b_ref[...]).reshape(_ROWS, 128)

    tx = head(wb0_ref, bb0_ref)
    ty = head(wb1_ref, bb1_ref)
    tw = head(wb2_ref, bb2_ref)
    th = head(wb3_ref, bb3_ref)
    sc_raw = head(wr_ref, br_ref)

    xa = xa_ref[...]
    ya = ya_ref[...]
    wa = wa_ref[...]
    ha = ha_ref[...]

    xc = xa + tx * wa
    yc = ya + ty * ha
    w = wa * jnp.exp(tw)
    h = ha * jnp.exp(th)
    x1 = jnp.clip(xc - w / 2, 0.0, _IMGW - 1.0)
    y1 = jnp.clip(yc - h / 2, 0.0, _IMGH - 1.0)
    x2 = jnp.clip(xc + w / 2, 0.0, _IMGW - 1.0)
    y2 = jnp.clip(yc + h / 2, 0.0, _IMGH - 1.0)
    wcl = x2 - x1
    hcl = y2 - y1
    valid = (wcl >= 1.0) & (hcl >= 1.0)
    xcc = (x1 + x2) / 2
    ycc = (y1 + y2) / 2
    bx1 = xcc - wcl / 2
    by1 = ycc - hcl / 2
    bx2 = xcc + wcl / 2
    by2 = ycc + hcl / 2
    areas = (bx2 - bx1) * (by2 - by1)

    neg_inf = jnp.float32(-jnp.inf)
    scores0 = jnp.where(valid, sc_raw, neg_inf)
    iota = (lax.broadcasted_iota(jnp.int32, (_ROWS, 128), 0) * 128
            + lax.broadcasted_iota(jnp.int32, (_ROWS, 128), 1))
    lane = lax.broadcasted_iota(jnp.int32, (1, 128), 1)

    dbg_ref[0] = scores0
    dbg_ref[1] = bx1
    dbg_ref[2] = by1
    dbg_ref[3] = bx2
    dbg_ref[4] = by2
    dbg_ref[5] = xcc

    def body(i, sc):
        m = jnp.max(sc)
        ok = m > neg_inf
        j = jnp.min(jnp.where(sc == m, iota, jnp.int32(1 << 30)))
        sel = (iota == j).astype(jnp.float32)
        xj1 = jnp.sum(sel * bx1)
        yj1 = jnp.sum(sel * by1)
        xj2 = jnp.sum(sel * bx2)
        yj2 = jnp.sum(sel * by2)
        aj = jnp.sum(sel * areas)
        xx1 = jnp.maximum(xj1, bx1)
        yy1 = jnp.maximum(yj1, by1)
        xx2 = jnp.minimum(xj2, bx2)
        yy2 = jnp.minimum(yj2, by2)
        inter = jnp.maximum(xx2 - xx1, 0.0) * jnp.maximum(yy2 - yy1, 0.0)
        iou = inter / (aj + areas - inter + 1e-10)
        supp = (iou > _NMS_T) | (iota == j)
        sc = jnp.where(ok & supp, neg_inf, sc)
        okf = jnp.where(ok, 1.0, 0.0).astype(jnp.float32)
        rxc = jnp.sum(sel * xcc) * okf
        ryc = jnp.sum(sel * ycc) * okf
        rw = jnp.sum(sel * wcl) * okf
        rh = jnp.sum(sel * hcl) * okf
        row = jnp.where(lane == 0, rxc,
              jnp.where(lane == 1, ryc,
              jnp.where(lane == 2, rw,
              jnp.where(lane == 3, rh,
              jnp.where(lane == 4, okf, 0.0))))).astype(jnp.float32)
        out_ref[pl.ds(i, 1), :] = row
        return sc

    lax.fori_loop(0, _MAXP, body, scores0)


def _roi_pool_body(fx_ref, fy_ref, m_ref, feat_ref, xg_ref, yg_ref, out_ref):
    tx = jnp.maximum(1.0 - jnp.abs(fx_ref[...] - xg_ref[...]), 0.0)
    ty = jnp.maximum(1.0 - jnp.abs(fy_ref[...] - yg_ref[...]), 0.0)
    s = tx * ty * m_ref[...]
    out_ref[...] = jnp.dot(s, feat_ref[...],
                           preferred_element_type=jnp.float32, precision=_HIGH)


def _stage1(features, W1, b1, W_box, b_box, W_rpn, b_rpn):
    ft = features[0].transpose(1, 2, 0)
    featf = ft.reshape(_P, 512)
    xp = jnp.pad(ft, ((1, 1), (1, 1), (0, 0)))
    cols = []
    for dy in range(3):
        for dx in range(3):
            cols.append(xp[dy:dy + 32, dx:dx + 32, :].reshape(_P, 512).T)
    xT = jnp.concatenate(cols, axis=0)
    w1T = W1.transpose(0, 2, 3, 1).reshape(256, 4608)
    b1c = b1[:, None]

    wb = [W_box[d::4, :, 0, 0] for d in range(4)]
    bb = [b_box[d::4][:, None] for d in range(4)]
    wr = W_rpn[:, :, 0, 0]
    br = b_rpn[:, None]

    anchors_wh = jnp.array(
        [[45, 90], [90, 45], [64, 64], [90, 180], [180, 90], [128, 128],
         [181, 362], [362, 181], [256, 256], [362, 724], [724, 362],
         [512, 512]], dtype=jnp.float32)
    xcg = _X0 + _SX * jnp.arange(_FW, dtype=jnp.float32)
    ycg = _Y0 + _SY * jnp.arange(_FH, dtype=jnp.float32)
    xa = jnp.broadcast_to(xcg[None, None, :], (_K, _FH, _FW)).reshape(_ROWS, 128)
    ya = jnp.broadcast_to(ycg[None, :, None], (_K, _FH, _FW)).reshape(_ROWS, 128)
    wa = jnp.broadcast_to(anchors_wh[:, 0][:, None, None], (_K, _FH, _FW)).reshape(_ROWS, 128)
    ha = jnp.broadcast_to(anchors_wh[:, 1][:, None, None], (_K, _FH, _FW)).reshape(_ROWS, 128)

    out1, _dbg = pl.pallas_call(
        _trunk_nms_body,
        out_shape=(jax.ShapeDtypeStruct((304, 128), jnp.float32),
                   jax.ShapeDtypeStruct((6, _ROWS, 128), jnp.float32)),
        interpret=_INTERPRET,
    )(xT, w1T, b1c, wb[0], wb[1], wb[2], wb[3], bb[0], bb[1], bb[2], bb[3],
      wr, br, xa, ya, wa, ha)
    return out1, _dbg, featf


def kernel(features, W1, b1, W_box, b_box, W_rpn, b_rpn):
    out1, _dbg, featf = _stage1(features, W1, b1, W_box, b_box, W_rpn, b_rpn)

    rois = out1[:_MAXP, :4]
    mask = out1[:_MAXP, 4]

    x1 = rois[:, 0] - rois[:, 2] / 2
    y1 = rois[:, 1] - rois[:, 3] / 2
    x2 = rois[:, 0] + rois[:, 2] / 2
    y2 = rois[:, 1] + rois[:, 3] / 2
    tyl = jnp.linspace(0.0, 1.0, _OH)
    txl = jnp.linspace(0.0, 1.0, _OW)
    ys = y1[:, None] + (y2 - y1)[:, None] * tyl[None, :]
    xs = x1[:, None] + (x2 - x1)[:, None] * txl[None, :]
    fy = jnp.clip(ys * (_FH - 1) / (_IMGH - 1), 0.0, _FH - 1.0)
    fx = jnp.clip(xs * (_FW - 1) / (_IMGW - 1), 0.0, _FW - 1.0)

    fyp = jnp.broadcast_to(fy[:, :, None], (_MAXP, _OH, _OW)).reshape(_NPIX)
    fxp = jnp.broadcast_to(fx[:, None, :], (_MAXP, _OH, _OW)).reshape(_NPIX)
    mp = jnp.broadcast_to(mask[:, None, None], (_MAXP, _OH, _OW)).reshape(_NPIX)
    pad = _NPAD - _NPIX
    fyp = jnp.pad(fyp, (0, pad))[:, None]
    fxp = jnp.pad(fxp, (0, pad))[:, None]
    mp = jnp.pad(mp, (0, pad))[:, None]

    xg = (jnp.arange(_P, dtype=jnp.float32) % _FW)[None, :]
    yg = (jnp.arange(_P, dtype=jnp.float32) // _FW)[None, :]

    out2 = pl.pallas_call(
        _roi_pool_body,
        grid=(_NPAD // _BLK,),
        in_specs=[
            pl.BlockSpec((_BLK, 1), lambda i: (i, 0)),
            pl.BlockSpec((_BLK, 1), lambda i: (i, 0)),
            pl.BlockSpec((_BLK, 1), lambda i: (i, 0)),
            pl.BlockSpec((_P, 512), lambda i: (0, 0)),
            pl.BlockSpec((1, _P), lambda i: (0, 0)),
            pl.BlockSpec((1, _P), lambda i: (0, 0)),
        ],
        out_specs=pl.BlockSpec((_BLK, 512), lambda i: (i, 0)),
        out_shape=jax.ShapeDtypeStruct((_NPAD, 512), jnp.float32),
        interpret=_INTERPRET,
    )(fxp, fyp, mp, featf, xg, yg)

    roi_feats = out2[:_NPIX].reshape(_MAXP, _OH, _OW, 512).transpose(0, 3, 1, 2)
    return roi_feats, rois


# speedup vs baseline: 12.3281x; 1.2244x over previous
import functools

import jax
import jax.numpy as jnp
from jax import lax
from jax.experimental import pallas as pl
from jax.experimental.pallas import tpu as pltpu

_INTERPRET = False

_K = 12
_FH = _FW = 32
_P = _FH * _FW
_NB = _K * _P
_ROWS = _NB // 128
_IMGW = 512.0
_IMGH = 512.0
_NMS_T = 0.7
_MAXP = 300
_OH = _OW = 7
_NPIX = _MAXP * _OH * _OW
_NPAD = 14720
_BLK = 1472
_X0 = 7.5
_Y0 = 7.5
_SX = 16.0
_SY = 16.0

_HIGH = lax.Precision.HIGHEST


def _trunk_nms_body(xT_ref, w1_ref, b1_ref,
                    wb0_ref, wb1_ref, wb2_ref, wb3_ref,
                    bb0_ref, bb1_ref, bb2_ref, bb3_ref,
                    wr_ref, br_ref,
                    xa_ref, ya_ref, wa_ref, ha_ref,
                    out_ref, dbg_ref):
    baseT = jnp.dot(w1_ref[...], xT_ref[...],
                    preferred_element_type=jnp.float32)
    baseT = jnp.maximum(baseT + b1_ref[...], 0.0)

    def head(w_ref, b_ref):
        h = jnp.dot(w_ref[...], baseT, preferred_element_type=jnp.float32)
        return (h + b_ref[...]).reshape(_ROWS, 128)

    tx = head(wb0_ref, bb0_ref)
    ty = head(wb1_ref, bb1_ref)
    tw = head(wb2_ref, bb2_ref)
    th = head(wb3_ref, bb3_ref)
    sc_raw = head(wr_ref, br_ref)

    xa = xa_ref[...]
    ya = ya_ref[...]
    wa = wa_ref[...]
    ha = ha_ref[...]

    xc = xa + tx * wa
    yc = ya + ty * ha
    w = wa * jnp.exp(tw)
    h = ha * jnp.exp(th)
    x1 = jnp.clip(xc - w / 2, 0.0, _IMGW - 1.0)
    y1 = jnp.clip(yc - h / 2, 0.0, _IMGH - 1.0)
    x2 = jnp.clip(xc + w / 2, 0.0, _IMGW - 1.0)
    y2 = jnp.clip(yc + h / 2, 0.0, _IMGH - 1.0)
    wcl = x2 - x1
    hcl = y2 - y1
    valid = (wcl >= 1.0) & (hcl >= 1.0)
    xcc = (x1 + x2) / 2
    ycc = (y1 + y2) / 2
    bx1 = xcc - wcl / 2
    by1 = ycc - hcl / 2
    bx2 = xcc + wcl / 2
    by2 = ycc + hcl / 2
    areas = (bx2 - bx1) * (by2 - by1)

    neg_inf = jnp.float32(-jnp.inf)
    scores0 = jnp.where(valid, sc_raw, neg_inf)
    iota = (lax.broadcasted_iota(jnp.int32, (_ROWS, 128), 0) * 128
            + lax.broadcasted_iota(jnp.int32, (_ROWS, 128), 1))
    lane = lax.broadcasted_iota(jnp.int32, (1, 128), 1)

    dbg_ref[0] = scores0
    dbg_ref[1] = bx1
    dbg_ref[2] = by1
    dbg_ref[3] = bx2
    dbg_ref[4] = by2
    dbg_ref[5] = xcc

    def body(i, sc):
        m = jnp.max(sc)
        ok = m > neg_inf
        j = jnp.min(jnp.where(sc == m, iota, jnp.int32(1 << 30)))
        sel = (iota == j).astype(jnp.float32)
        xj1 = jnp.sum(sel * bx1)
        yj1 = jnp.sum(sel * by1)
        xj2 = jnp.sum(sel * bx2)
        yj2 = jnp.sum(sel * by2)
        aj = jnp.sum(sel * areas)
        xx1 = jnp.maximum(xj1, bx1)
        yy1 = jnp.maximum(yj1, by1)
        xx2 = jnp.minimum(xj2, bx2)
        yy2 = jnp.minimum(yj2, by2)
        inter = jnp.maximum(xx2 - xx1, 0.0) * jnp.maximum(yy2 - yy1, 0.0)
        iou = inter / (aj + areas - inter + 1e-10)
        supp = (iou > _NMS_T) | (iota == j)
        sc = jnp.where(ok & supp, neg_inf, sc)
        okf = jnp.where(ok, 1.0, 0.0).astype(jnp.float32)
        rxc = jnp.sum(sel * xcc) * okf
        ryc = jnp.sum(sel * ycc) * okf
        rw = jnp.sum(sel * wcl) * okf
        rh = jnp.sum(sel * hcl) * okf
        row = jnp.where(lane == 0, rxc,
              jnp.where(lane == 1, ryc,
              jnp.where(lane == 2, rw,
              jnp.where(lane == 3, rh,
              jnp.where(lane == 4, okf, 0.0))))).astype(jnp.float32)
        out_ref[pl.ds(i, 1), :] = row
        return sc

    lax.fori_loop(0, _MAXP, body, scores0)


def _roi_pool_body(fx_ref, fy_ref, m_ref, feat_ref, xg_ref, yg_ref, out_ref):
    tx = jnp.maximum(1.0 - jnp.abs(fx_ref[...] - xg_ref[...]), 0.0)
    ty = jnp.maximum(1.0 - jnp.abs(fy_ref[...] - yg_ref[...]), 0.0)
    s = tx * ty * m_ref[...]
    out_ref[...] = jnp.dot(s, feat_ref[...],
                           preferred_element_type=jnp.float32)


def _stage1(features, W1, b1, W_box, b_box, W_rpn, b_rpn):
    ft = features[0].transpose(1, 2, 0)
    featf = ft.reshape(_P, 512)
    xp = jnp.pad(ft, ((1, 1), (1, 1), (0, 0)))
    cols = []
    for dy in range(3):
        for dx in range(3):
            cols.append(xp[dy:dy + 32, dx:dx + 32, :].reshape(_P, 512).T)
    xT = jnp.concatenate(cols, axis=0)
    w1T = W1.transpose(0, 2, 3, 1).reshape(256, 4608)
    b1c = b1[:, None]

    wb = [W_box[d::4, :, 0, 0] for d in range(4)]
    bb = [b_box[d::4][:, None] for d in range(4)]
    wr = W_rpn[:, :, 0, 0]
    br = b_rpn[:, None]

    anchors_wh = jnp.array(
        [[45, 90], [90, 45], [64, 64], [90, 180], [180, 90], [128, 128],
         [181, 362], [362, 181], [256, 256], [362, 724], [724, 362],
         [512, 512]], dtype=jnp.float32)
    xcg = _X0 + _SX * jnp.arange(_FW, dtype=jnp.float32)
    ycg = _Y0 + _SY * jnp.arange(_FH, dtype=jnp.float32)
    xa = jnp.broadcast_to(xcg[None, None, :], (_K, _FH, _FW)).reshape(_ROWS, 128)
    ya = jnp.broadcast_to(ycg[None, :, None], (_K, _FH, _FW)).reshape(_ROWS, 128)
    wa = jnp.broadcast_to(anchors_wh[:, 0][:, None, None], (_K, _FH, _FW)).reshape(_ROWS, 128)
    ha = jnp.broadcast_to(anchors_wh[:, 1][:, None, None], (_K, _FH, _FW)).reshape(_ROWS, 128)

    out1, _dbg = pl.pallas_call(
        _trunk_nms_body,
        out_shape=(jax.ShapeDtypeStruct((304, 128), jnp.float32),
                   jax.ShapeDtypeStruct((6, _ROWS, 128), jnp.float32)),
        interpret=_INTERPRET,
    )(xT, w1T, b1c, wb[0], wb[1], wb[2], wb[3], bb[0], bb[1], bb[2], bb[3],
      wr, br, xa, ya, wa, ha)
    return out1, _dbg, featf


def kernel(features, W1, b1, W_box, b_box, W_rpn, b_rpn):
    out1, _dbg, featf = _stage1(features, W1, b1, W_box, b_box, W_rpn, b_rpn)

    rois = out1[:_MAXP, :4]
    mask = out1[:_MAXP, 4]

    x1 = rois[:, 0] - rois[:, 2] / 2
    y1 = rois[:, 1] - rois[:, 3] / 2
    x2 = rois[:, 0] + rois[:, 2] / 2
    y2 = rois[:, 1] + rois[:, 3] / 2
    tyl = jnp.linspace(0.0, 1.0, _OH)
    txl = jnp.linspace(0.0, 1.0, _OW)
    ys = y1[:, None] + (y2 - y1)[:, None] * tyl[None, :]
    xs = x1[:, None] + (x2 - x1)[:, None] * txl[None, :]
    fy = jnp.clip(ys * (_FH - 1) / (_IMGH - 1), 0.0, _FH - 1.0)
    fx = jnp.clip(xs * (_FW - 1) / (_IMGW - 1), 0.0, _FW - 1.0)

    fyp = jnp.broadcast_to(fy[:, :, None], (_MAXP, _OH, _OW)).reshape(_NPIX)
    fxp = jnp.broadcast_to(fx[:, None, :], (_MAXP, _OH, _OW)).reshape(_NPIX)
    mp = jnp.broadcast_to(mask[:, None, None], (_MAXP, _OH, _OW)).reshape(_NPIX)
    pad = _NPAD - _NPIX
    fyp = jnp.pad(fyp, (0, pad))[:, None]
    fxp = jnp.pad(fxp, (0, pad))[:, None]
    mp = jnp.pad(mp, (0, pad))[:, None]

    xg = (jnp.arange(_P, dtype=jnp.float32) % _FW)[None, :]
    yg = (jnp.arange(_P, dtype=jnp.float32) // _FW)[None, :]

    out2 = pl.pallas_call(
        _roi_pool_body,
        grid=(_NPAD // _BLK,),
        in_specs=[
            pl.BlockSpec((_BLK, 1), lambda i: (i, 0)),
            pl.BlockSpec((_BLK, 1), lambda i: (i, 0)),
            pl.BlockSpec((_BLK, 1), lambda i: (i, 0)),
            pl.BlockSpec((_P, 512), lambda i: (0, 0)),
            pl.BlockSpec((1, _P), lambda i: (0, 0)),
            pl.BlockSpec((1, _P), lambda i: (0, 0)),
        ],
        out_specs=pl.BlockSpec((_BLK, 512), lambda i: (i, 0)),
        out_shape=jax.ShapeDtypeStruct((_NPAD, 512), jnp.float32),
        interpret=_INTERPRET,
    )(fxp, fyp, mp, featf, xg, yg)

    roi_feats = out2[:_NPIX].reshape(_MAXP, _OH, _OW, 512).transpose(0, 3, 1, 2)
    return roi_feats, rois
